# baseline (speedup 1.0000x reference)
import jax
import jax.numpy as jnp
from jax.experimental import pallas as pl
from jax.experimental.pallas import tpu as pltpu

_B = 1024
_S = 4096
_I = 256
_BJ = 512
_NJ = _S // _BJ


def _esn_tile(inputs_ref, states_ref, win_ref, wres_ref, out_ref, zbuf_ref):
    t = pl.program_id(0)
    out_ref[...] = jnp.tanh(zbuf_ref[(t + 1) % 2])
    z = jnp.dot(states_ref[...], wres_ref[...],
                preferred_element_type=jnp.float32)
    z = z + jnp.dot(inputs_ref[...], win_ref[...],
                    preferred_element_type=jnp.float32)
    zbuf_ref[t % 2] = z


def kernel(inputs, states, Win, Wres):
    clamp = lambda t: jnp.minimum(t, _NJ - 1)
    prev = lambda t: jnp.maximum(t - 1, 0)
    return pl.pallas_call(
        _esn_tile,
        grid=(_NJ + 1,),
        in_specs=[
            pl.BlockSpec((_B, _I), lambda t: (0, 0)),
            pl.BlockSpec((_B, _S), lambda t: (0, 0)),
            pl.BlockSpec((_I, _BJ), lambda t: (0, clamp(t))),
            pl.BlockSpec((_S, _BJ), lambda t: (0, clamp(t))),
        ],
        out_specs=pl.BlockSpec((_B, _BJ), lambda t: (0, prev(t))),
        out_shape=jax.ShapeDtypeStruct((_B, _S), jnp.float32),
        scratch_shapes=[pltpu.VMEM((2, _B, _BJ), jnp.float32)],
    )(inputs, states, Win, Wres)


# speedup vs baseline: 1.2293x; 1.2293x over previous
import jax
import jax.numpy as jnp
from jax.experimental import pallas as pl

_B = 1024
_S = 4096
_I = 256
_BJ = 512
_NJ = _S // _BJ


def _esn_tile(inputs_ref, states_ref, win_ref, wres_ref, out_ref):
    t = pl.program_id(0)
    z = jnp.dot(states_ref[...], wres_ref[...],
                preferred_element_type=jnp.float32)
    z = z + jnp.dot(inputs_ref[...], win_ref[...],
                    preferred_element_type=jnp.float32)
    cand = jnp.tanh(z)
    sj = states_ref[:, pl.ds(t * _BJ, _BJ)]
    out_ref[...] = sj + (cand - sj)


def kernel(inputs, states, Win, Wres):
    return pl.pallas_call(
        _esn_tile,
        grid=(_NJ,),
        in_specs=[
            pl.BlockSpec((_B, _I), lambda t: (0, 0)),
            pl.BlockSpec((_B, _S), lambda t: (0, 0)),
            pl.BlockSpec((_I, _BJ), lambda t: (0, t)),
            pl.BlockSpec((_S, _BJ), lambda t: (0, t),
                         pipeline_mode=pl.Buffered(buffer_count=2)),
        ],
        out_specs=pl.BlockSpec((_B, _BJ), lambda t: (0, t),
                               pipeline_mode=pl.Buffered(buffer_count=2)),
        out_shape=jax.ShapeDtypeStruct((_B, _S), jnp.float32),
    )(inputs, states, Win, Wres)
